# Initial kernel scaffold; baseline (speedup 1.0000x reference)
import jax
import jax.numpy as jnp
from jax.experimental import pallas as pl


def kernel(stu_id, input_exercise, inut_word, inut_format, inut_section, inut_wordlen, inut_cefr, input_knowledge_point, student_emb, k_difficulty_emb, e_difficulty_emb):
    raise NotImplementedError("write your pallas kernel here")


# speedup vs baseline: 1.1228x; 1.1228x over previous
import functools

import jax
import jax.numpy as jnp
from jax import lax
from jax.experimental import pallas as pl
from jax.experimental.pallas import tpu as pltpu
from jax.experimental.pallas import tpu_sc as plsc

B = 16384

_info = plsc.get_sparse_core_info()
_NC, _NS, _L = _info.num_cores, _info.num_subcores, _info.num_lanes
_NW = _NC * _NS
_BPW = B // _NW

_mesh = plsc.VectorSubcoreMesh(core_axis_name="c", subcore_axis_name="s")


@functools.partial(
    pl.kernel,
    mesh=_mesh,
    out_type=jax.ShapeDtypeStruct((B,), jnp.float32),
    scratch_types=[
        pltpu.VMEM((_BPW,), jnp.int32),
        pltpu.VMEM((_BPW,), jnp.int32),
        pltpu.VMEM((_BPW,), jnp.float32),
        pltpu.VMEM((_BPW,), jnp.float32),
        pltpu.VMEM((_BPW,), jnp.float32),
        pltpu.VMEM((_BPW,), jnp.float32),
        pltpu.SemaphoreType.DMA,
        pltpu.SemaphoreType.DMA,
        pltpu.SemaphoreType.DMA,
    ],
)
def _sc_forward(stu_hbm, ex_hbm, stab_hbm, ktab_hbm, etab_hbm, out_hbm,
                sidx, eidx, sv, kv, ev, ov, sem_s, sem_k, sem_e):
    wid = lax.axis_index("s") * _NC + lax.axis_index("c")
    base = wid * _BPW
    pltpu.sync_copy(stu_hbm.at[pl.ds(base, _BPW)], sidx)
    pltpu.sync_copy(ex_hbm.at[pl.ds(base, _BPW)], eidx)
    cs = pltpu.async_copy(stab_hbm.at[sidx], sv, sem_s)
    ck = pltpu.async_copy(ktab_hbm.at[eidx], kv, sem_k)
    ce = pltpu.async_copy(etab_hbm.at[eidx], ev, sem_e)
    cs.wait()
    ck.wait()
    ce.wait()
    for i in range(_BPW // _L):
        sl = pl.ds(i * _L, _L)
        s = sv[sl]
        k0 = kv[sl]
        e0 = ev[sl]
        stat = 8.0 / (1.0 + jnp.exp(-s)) - 4.0
        kd = 8.0 / (1.0 + jnp.exp(-k0)) - 4.0
        ed = 2.0 / (1.0 + jnp.exp(-e0))
        x = jnp.exp(-1.7 * ed * (stat - kd))
        ov[sl] = 1.0 / (1.0 + jnp.exp(-x))
    pltpu.sync_copy(ov, out_hbm.at[pl.ds(base, _BPW)])


def kernel(stu_id, input_exercise, inut_word, inut_format, inut_section,
           inut_wordlen, inut_cefr, input_knowledge_point,
           student_emb, k_difficulty_emb, e_difficulty_emb):
    del inut_word, inut_format, inut_section, inut_wordlen, inut_cefr
    del input_knowledge_point
    return _sc_forward(
        stu_id.astype(jnp.int32),
        input_exercise.astype(jnp.int32),
        student_emb.reshape(-1),
        k_difficulty_emb.reshape(-1),
        e_difficulty_emb.reshape(-1),
    )


# speedup vs baseline: 1.1400x; 1.0153x over previous
import functools

import jax
import jax.numpy as jnp
from jax import lax
from jax.experimental import pallas as pl
from jax.experimental.pallas import tpu as pltpu
from jax.experimental.pallas import tpu_sc as plsc

B = 16384

_info = plsc.get_sparse_core_info()
_NC, _NS, _L = _info.num_cores, _info.num_subcores, _info.num_lanes
_NW = _NC * _NS
_BPW = B // _NW
_NH = 2
_H = _BPW // _NH


def _forward_chunk(s, k0, e0):
    s = jnp.minimum(jnp.maximum(s, -60.0), 60.0)
    k0 = jnp.minimum(jnp.maximum(k0, -60.0), 60.0)
    e0 = jnp.minimum(jnp.maximum(e0, -60.0), 60.0)
    a = jnp.exp(-s)
    b = jnp.exp(-k0)
    c = jnp.exp(-e0)
    num = b - a
    den = (1.0 + a) * (1.0 + b) * (1.0 + c)
    t = 27.2 * num / den
    x = jnp.exp(-t)
    return 1.0 / (1.0 + jnp.exp(-x))


@functools.partial(
    pl.kernel,
    mesh=plsc.VectorSubcoreMesh(core_axis_name="c", subcore_axis_name="s"),
    out_type=jax.ShapeDtypeStruct((B,), jnp.float32),
    scratch_types=(
        [pltpu.VMEM((_H,), jnp.int32) for _ in range(2 * _NH)]
        + [pltpu.VMEM((_H,), jnp.float32) for _ in range(4 * _NH)]
        + [pltpu.SemaphoreType.DMA for _ in range(3 + _NH)]
    ),
)
def _sc_forward(stu_hbm, ex_hbm, stab_hbm, ktab_hbm, etab_hbm, out_hbm, *scr):
    sidx = scr[0:_NH]
    eidx = scr[_NH:2 * _NH]
    sv = scr[2 * _NH:3 * _NH]
    kv = scr[3 * _NH:4 * _NH]
    ev = scr[4 * _NH:5 * _NH]
    ov = scr[5 * _NH:6 * _NH]
    sem_i, sem_o = scr[6 * _NH], scr[6 * _NH + 1]
    sem_g = scr[6 * _NH + 2:]
    wid = lax.axis_index("s") * _NC + lax.axis_index("c")
    base = wid * _BPW
    idx_copies = []
    for h in range(_NH):
        src = pl.ds(base + h * _H, _H)
        idx_copies.append(pltpu.async_copy(stu_hbm.at[src], sidx[h], sem_i))
        idx_copies.append(pltpu.async_copy(ex_hbm.at[src], eidx[h], sem_i))
    for cp in idx_copies:
        cp.wait()
    gathers = []
    for h in range(_NH):
        gathers.append((
            pltpu.async_copy(stab_hbm.at[sidx[h]], sv[h], sem_g[h]),
            pltpu.async_copy(ktab_hbm.at[eidx[h]], kv[h], sem_g[h]),
            pltpu.async_copy(etab_hbm.at[eidx[h]], ev[h], sem_g[h]),
        ))
    out_copies = []
    for h in range(_NH):
        for cp in gathers[h]:
            cp.wait()
        for i in range(_H // _L):
            sl = pl.ds(i * _L, _L)
            ov[h][sl] = _forward_chunk(sv[h][sl], kv[h][sl], ev[h][sl])
        out_copies.append(pltpu.async_copy(
            ov[h], out_hbm.at[pl.ds(base + h * _H, _H)], sem_o))
    for cp in out_copies:
        cp.wait()


def kernel(stu_id, input_exercise, inut_word, inut_format, inut_section,
           inut_wordlen, inut_cefr, input_knowledge_point,
           student_emb, k_difficulty_emb, e_difficulty_emb):
    del inut_word, inut_format, inut_section, inut_wordlen, inut_cefr
    del input_knowledge_point
    return _sc_forward(
        stu_id.astype(jnp.int32),
        input_exercise.astype(jnp.int32),
        student_emb.reshape(-1),
        k_difficulty_emb.reshape(-1),
        e_difficulty_emb.reshape(-1),
    )
